# Initial kernel scaffold; baseline (speedup 1.0000x reference)
import jax
import jax.numpy as jnp
from jax.experimental import pallas as pl


def kernel(s, v, coord, edge_index, eW1, eb1, eW2, eb2, evW1, evb1, evW2, evb2, uW1, ub1, uW2, ub2, uvW1, uvb1, uvW2, uvb2):
    raise NotImplementedError("write your pallas kernel here")


# speedup vs baseline: 3.6628x; 3.6628x over previous
import jax
import jax.numpy as jnp
from jax import lax
from jax.experimental import pallas as pl
from jax.experimental.pallas import tpu as pltpu
from jax.experimental.pallas import tpu_sc as plsc

N, E, DS = 10000, 320000, 128
NC, NS, L = 2, 16, 16
EPT = E // NS
EPW = E // (NC * NS)
CH = 80
CCH = 40
BK = 10
NP = 10240
RPT = NP // NS


def _silu(x):
    return x / (1.0 + jnp.exp(-x))


def _prep_body(s_ref, w1_ref, b1_ref, coord_ref, evw1_ref, ta_ref, tb_ref,
               cps_ref):
    s = s_ref[...]
    a = jnp.dot(s, w1_ref[:DS, :], preferred_element_type=jnp.float32) + b1_ref[...]
    b = jnp.dot(s, w1_ref[DS:, :], preferred_element_type=jnp.float32)
    ta_ref[0] = a[:, :DS]
    ta_ref[1] = a[:, DS:]
    tb_ref[0] = b[:, :DS]
    tb_ref[1] = b[:, DS:]
    cps_ref[...] = coord_ref[...] * evw1_ref[0, 0]


def _prep(s, eW1, eb1_2d, coord, evW1):
    out = jax.ShapeDtypeStruct((2, N, DS), jnp.float32)
    outc = jax.ShapeDtypeStruct((N, 3), jnp.float32)
    return pl.pallas_call(_prep_body, out_shape=[out, out, outc])(
        s, eW1, eb1_2d, coord, evW1)


def _edge_pipeline(ch, nblk, ebase, rowh, colh, src_a, src_b, acc,
                   rblk, cblk, sidx0, sidx1, a0, b0, a1, b1,
                   sga0, sgb0, sga1, sgb1, ssc0, ssc1, compute):
    def sidx_fill(k, sidx):
        def cp(j, carry):
            sidx[pl.ds(j * L, L)] = rblk[pl.ds(k * ch + j * L, L)]
            return carry
        lax.fori_loop(0, ch // L, cp, 0)

    def g_issue(k, aS, bS, sa, sb):
        ga = pltpu.async_copy(src_a(rblk.at[pl.ds(k * ch, ch)]), aS, sa)
        gb = pltpu.async_copy(src_b(cblk.at[pl.ds(k * ch, ch)]), bS, sb)
        return ga, gb

    def scat(sidx, aS, sem):
        return pltpu.async_copy(aS, acc.at[sidx], sem, add=True)

    def block(bi, carry):
        base = ebase + bi * (BK * ch)
        pltpu.sync_copy(rowh.at[pl.ds(base, BK * ch)], rblk)
        pltpu.sync_copy(colh.at[pl.ds(base, BK * ch)], cblk)
        g0 = g_issue(0, a0, b0, sga0, sgb0)
        sc0 = sc1 = None
        for p in range(BK // 2):
            k0 = 2 * p
            k1 = 2 * p + 1
            if sc1 is not None:
                sc1.wait()
            g1 = g_issue(k1, a1, b1, sga1, sgb1)
            g0[0].wait()
            g0[1].wait()
            compute(a0, b0)
            sidx_fill(k0, sidx0)
            sc0 = scat(sidx0, a0, ssc0)
            g1[0].wait()
            g1[1].wait()
            compute(a1, b1)
            sidx_fill(k1, sidx1)
            sc1 = scat(sidx1, a1, ssc1)
            if p < BK // 2 - 1:
                sc0.wait()
                g0 = g_issue(k0 + 2, a0, b0, sga0, sgb0)
        sc0.wait()
        sc1.wait()
        return carry
    lax.fori_loop(0, nblk, block, 0)


def _sch_body(ta, tb, rowh, colh, zh,
              hout,
              H, rblk, cblk, sidx0, sidx1, a0, b0, a1, b1,
              sga0, sgb0, sga1, sgb1, ssc0, ssc1):
    c = lax.axis_index("c")
    t = lax.axis_index("s")
    rows0 = t * RPT

    pltpu.sync_copy(zh, H.at[pl.ds(rows0, RPT)])
    plsc.subcore_barrier()

    tac = ta.at[c]
    tbc = tb.at[c]

    def compute(aS, bS):
        def cj(j, carry):
            for k8 in range(DS // L):
                sl = pl.ds(k8 * L, L)
                x = aS[j, sl] + bS[j, sl]
                aS[j, sl] = x / (1.0 + jnp.exp(-x))
            return carry
        lax.fori_loop(0, CH, cj, 0)

    _edge_pipeline(CH, EPT // (BK * CH), t * EPT, rowh, colh,
                   lambda idx: tac.at[idx], lambda idx: tbc.at[idx], H,
                   rblk, cblk, sidx0, sidx1, a0, b0, a1, b1,
                   sga0, sgb0, sga1, sgb1, ssc0, ssc1, compute)

    plsc.subcore_barrier()
    pltpu.sync_copy(H.at[pl.ds(rows0, RPT)], hout.at[c, pl.ds(rows0, RPT)])


def _sch_call(ta3, tb3, row, col, zh):
    mesh = plsc.VectorSubcoreMesh(core_axis_name="c", subcore_axis_name="s")
    kern = pl.kernel(
        _sch_body,
        mesh=mesh,
        compiler_params=pltpu.CompilerParams(needs_layout_passes=False),
        out_type=[jax.ShapeDtypeStruct((NC, NP, DS), jnp.float32)],
        scratch_types=[
            pltpu.VMEM_SHARED((NP, DS), jnp.float32),
            pltpu.VMEM((BK * CH,), jnp.int32),
            pltpu.VMEM((BK * CH,), jnp.int32),
            pltpu.VMEM((CH,), jnp.int32),
            pltpu.VMEM((CH,), jnp.int32),
            pltpu.VMEM((CH, DS), jnp.float32),
            pltpu.VMEM((CH, DS), jnp.float32),
            pltpu.VMEM((CH, DS), jnp.float32),
            pltpu.VMEM((CH, DS), jnp.float32),
            pltpu.SemaphoreType.DMA,
            pltpu.SemaphoreType.DMA,
            pltpu.SemaphoreType.DMA,
            pltpu.SemaphoreType.DMA,
            pltpu.SemaphoreType.DMA,
            pltpu.SemaphoreType.DMA,
        ],
    )
    return kern(ta3, tb3, row, col, zh)


def _scc_body(cp128, rowh, colh, b1h, zc,
              cout,
              C, rblk, cblk, sidx0, sidx1, a0, b0, a1, b1, pb,
              sga0, sgb0, sga1, sgb1, ssc0, ssc1):
    c = lax.axis_index("c")
    t = lax.axis_index("s")
    rows0 = t * RPT

    pltpu.sync_copy(zc, C.at[pl.ds(rows0, RPT)])
    pltpu.sync_copy(b1h, pb)
    plsc.subcore_barrier()

    lane = jax.lax.iota(jnp.int32, L)
    m_cv = lane < 3
    m_deg = lane == 3
    bias1 = pb[...]
    cpc = cp128.at[c]

    def compute(aS, bS):
        def cj(j, carry):
            sl = pl.ds(0, L)
            d = aS[j, sl] - bS[j, sl]
            sv = _silu(d + bias1)
            aS[j, sl] = jnp.where(m_cv, sv, jnp.where(m_deg, 1.0, 0.0))
            return carry
        lax.fori_loop(0, CCH, cj, 0)

    w = t * NC + c
    _edge_pipeline(CCH, EPW // (BK * CCH), w * EPW, rowh, colh,
                   lambda idx: cpc.at[idx], lambda idx: cpc.at[idx], C,
                   rblk, cblk, sidx0, sidx1, a0, b0, a1, b1,
                   sga0, sgb0, sga1, sgb1, ssc0, ssc1, compute)

    plsc.subcore_barrier()
    pltpu.sync_copy(C.at[pl.ds(rows0, RPT)], cout.at[c, pl.ds(rows0, RPT)])


def _scc_call(cp128, row, col, b1v, zc):
    mesh = plsc.VectorSubcoreMesh(core_axis_name="c", subcore_axis_name="s")
    kern = pl.kernel(
        _scc_body,
        mesh=mesh,
        compiler_params=pltpu.CompilerParams(needs_layout_passes=False),
        out_type=[jax.ShapeDtypeStruct((NC, NP, DS), jnp.float32)],
        scratch_types=[
            pltpu.VMEM_SHARED((NP, DS), jnp.float32),
            pltpu.VMEM((BK * CCH,), jnp.int32),
            pltpu.VMEM((BK * CCH,), jnp.int32),
            pltpu.VMEM((CCH,), jnp.int32),
            pltpu.VMEM((CCH,), jnp.int32),
            pltpu.VMEM((CCH, DS), jnp.float32),
            pltpu.VMEM((CCH, DS), jnp.float32),
            pltpu.VMEM((CCH, DS), jnp.float32),
            pltpu.VMEM((CCH, DS), jnp.float32),
            pltpu.VMEM((L,), jnp.float32),
            pltpu.SemaphoreType.DMA,
            pltpu.SemaphoreType.DMA,
            pltpu.SemaphoreType.DMA,
            pltpu.SemaphoreType.DMA,
            pltpu.SemaphoreType.DMA,
            pltpu.SemaphoreType.DMA,
        ],
    )
    return kern(cp128, row, col, b1v, zc)


def _post_body(h_ref, c_ref, w2_ref, b2_ref, uW1_ref, ub1_ref, uW2_ref,
               ub2_ref, s_ref, v_ref, scal_ref, so_ref, vo_ref):
    cvd = c_ref[0, :, :L] + c_ref[1, :, :L]
    deg = cvd[:, 3:4]
    agg_s = (jnp.dot(h_ref[0], w2_ref[:DS, :], preferred_element_type=jnp.float32)
             + jnp.dot(h_ref[1], w2_ref[DS:, :], preferred_element_type=jnp.float32)
             + deg * b2_ref[...])
    tt = _silu(jnp.dot(agg_s, uW1_ref[...], preferred_element_type=jnp.float32)
               + ub1_ref[...])
    so_ref[...] = (s_ref[...] + jnp.dot(tt, uW2_ref[...],
                                        preferred_element_type=jnp.float32)
                   + ub2_ref[...])

    evW2 = scal_ref[0, 0]
    evb2 = scal_ref[0, 1]
    uvW1 = scal_ref[0, 2]
    uvb1 = scal_ref[0, 3]
    uvW2 = scal_ref[0, 4]
    uvb2 = scal_ref[0, 5]
    cv = cvd[:, :3]
    agg_v = cv * evW2 + deg * evb2
    vo_ref[...] = v_ref[...] + _silu(agg_v * uvW1 + uvb1) * uvW2 + uvb2


def _post(hout, cout, eW2, eb2_2d, uW1, ub1_2d, uW2, ub2_2d, s, v2, scal):
    return pl.pallas_call(
        _post_body,
        out_shape=[
            jax.ShapeDtypeStruct((N, DS), jnp.float32),
            jax.ShapeDtypeStruct((N, 3), jnp.float32),
        ],
    )(hout, cout, eW2, eb2_2d, uW1, ub1_2d, uW2, ub2_2d, s, v2, scal)


def kernel(s, v, coord, edge_index, eW1, eb1, eW2, eb2, evW1, evb1, evW2,
           evb2, uW1, ub1, uW2, ub2, uvW1, uvb1, uvW2, uvb2):
    row = edge_index[0]
    col = edge_index[1]

    ta3, tb3, cps = _prep(s, eW1, eb1[None, :], coord, evW1)

    cp128 = jnp.broadcast_to(jnp.pad(cps, ((0, 0), (0, DS - 3)))[None],
                             (NC, N, DS))
    b1v = jnp.full((L,), evb1[0], jnp.float32)
    zh = jnp.zeros((RPT, DS), jnp.float32)

    hout = _sch_call(ta3, tb3, row, col, zh)[0]
    cout = _scc_call(cp128, row, col, b1v, zh)[0]

    scal = jnp.stack([evW2[0, 0], evb2[0], uvW1[0, 0], uvb1[0], uvW2[0, 0],
                      uvb2[0], jnp.float32(0), jnp.float32(0)])[None, :]
    s_out, v_out = _post(hout[:, :N], cout[:, :N], eW2, eb2[None, :], uW1,
                         ub1[None, :], uW2, ub2[None, :], s, v.reshape(N, 3),
                         scal)
    return (s_out, v_out.reshape(N, 3, 1))


# speedup vs baseline: 4.7824x; 1.3057x over previous
import jax
import jax.numpy as jnp
from jax import lax
from jax.experimental import pallas as pl
from jax.experimental.pallas import tpu as pltpu
from jax.experimental.pallas import tpu_sc as plsc

N, E, DS = 10000, 320000, 128
NC, NS, L = 2, 16, 16
EPT = E // NS
EPW = E // (NC * NS)
CH = 80
CCH = 40
BK = 10
NP = 10240
RPT = NP // NS


def _silu(x):
    return x / (1.0 + jnp.exp(-x))


def _prep_body(s_ref, w1_ref, b1_ref, coord_ref, evw1_ref, ta_ref, tb_ref,
               cps_ref):
    s = s_ref[...]
    a = jnp.dot(s, w1_ref[:DS, :], preferred_element_type=jnp.float32) + b1_ref[...]
    b = jnp.dot(s, w1_ref[DS:, :], preferred_element_type=jnp.float32)
    ta_ref[0] = a[:, :DS]
    ta_ref[1] = a[:, DS:]
    tb_ref[0] = b[:, :DS]
    tb_ref[1] = b[:, DS:]
    cps_ref[...] = coord_ref[...] * evw1_ref[0, 0]


def _prep(s, eW1, eb1_2d, coord, evW1):
    out = jax.ShapeDtypeStruct((2, N, DS), jnp.float32)
    outc = jax.ShapeDtypeStruct((N, 3), jnp.float32)
    return pl.pallas_call(_prep_body, out_shape=[out, out, outc])(
        s, eW1, eb1_2d, coord, evW1)


def _edge_pipeline(ch, nblk, ebase, rowh, colh, src_a, src_b, acc,
                   rblk, cblk, sidx0, sidx1, a0, b0, a1, b1,
                   sga0, sgb0, sga1, sgb1, ssc0, ssc1, compute):
    def sidx_fill(k, sidx):
        def cp(j, carry):
            sidx[pl.ds(j * L, L)] = rblk[pl.ds(k * ch + j * L, L)]
            return carry
        lax.fori_loop(0, ch // L, cp, 0)

    def g_issue(k, aS, bS, sa, sb):
        ga = pltpu.async_copy(src_a(rblk.at[pl.ds(k * ch, ch)]), aS, sa)
        gb = pltpu.async_copy(src_b(cblk.at[pl.ds(k * ch, ch)]), bS, sb)
        return ga, gb

    def scat(sidx, aS, sem):
        return pltpu.async_copy(aS, acc.at[sidx], sem, add=True)

    def block(bi, carry):
        base = ebase + bi * (BK * ch)
        pltpu.sync_copy(rowh.at[pl.ds(base, BK * ch)], rblk)
        pltpu.sync_copy(colh.at[pl.ds(base, BK * ch)], cblk)
        g0 = g_issue(0, a0, b0, sga0, sgb0)
        sc0 = sc1 = None
        for p in range(BK // 2):
            k0 = 2 * p
            k1 = 2 * p + 1
            if sc1 is not None:
                sc1.wait()
            g1 = g_issue(k1, a1, b1, sga1, sgb1)
            g0[0].wait()
            g0[1].wait()
            compute(a0, b0)
            sidx_fill(k0, sidx0)
            sc0 = scat(sidx0, a0, ssc0)
            g1[0].wait()
            g1[1].wait()
            compute(a1, b1)
            sidx_fill(k1, sidx1)
            sc1 = scat(sidx1, a1, ssc1)
            if p < BK // 2 - 1:
                sc0.wait()
                g0 = g_issue(k0 + 2, a0, b0, sga0, sgb0)
        sc0.wait()
        sc1.wait()
        return carry
    lax.fori_loop(0, nblk, block, 0)


def _sch_body(ta, tb, rowh, colh, zh,
              hout,
              H, rblk, cblk, sidx0, sidx1, a0, b0, a1, b1,
              sga0, sgb0, sga1, sgb1, ssc0, ssc1):
    c = lax.axis_index("c")
    t = lax.axis_index("s")
    rows0 = t * RPT

    pltpu.sync_copy(zh, H.at[pl.ds(rows0, RPT)])
    plsc.subcore_barrier()

    tac = ta.at[c]
    tbc = tb.at[c]

    def compute(aS, bS):
        def cj(j2, carry):
            for r in range(2):
                j = j2 * 2 + r
                for k8 in range(DS // L):
                    sl = pl.ds(k8 * L, L)
                    x = aS[j, sl] + bS[j, sl]
                    aS[j, sl] = x / (1.0 + jnp.exp(-x))
            return carry
        lax.fori_loop(0, CH // 2, cj, 0)

    _edge_pipeline(CH, EPT // (BK * CH), t * EPT, rowh, colh,
                   lambda idx: tac.at[idx], lambda idx: tbc.at[idx], H,
                   rblk, cblk, sidx0, sidx1, a0, b0, a1, b1,
                   sga0, sgb0, sga1, sgb1, ssc0, ssc1, compute)

    plsc.subcore_barrier()
    pltpu.sync_copy(H.at[pl.ds(rows0, RPT)], hout.at[c, pl.ds(rows0, RPT)])


def _sch_call(ta3, tb3, row, col, zh):
    mesh = plsc.VectorSubcoreMesh(core_axis_name="c", subcore_axis_name="s")
    kern = pl.kernel(
        _sch_body,
        mesh=mesh,
        compiler_params=pltpu.CompilerParams(needs_layout_passes=False),
        out_type=[jax.ShapeDtypeStruct((NC, NP, DS), jnp.float32)],
        scratch_types=[
            pltpu.VMEM_SHARED((NP, DS), jnp.float32),
            pltpu.VMEM((BK * CH,), jnp.int32),
            pltpu.VMEM((BK * CH,), jnp.int32),
            pltpu.VMEM((CH,), jnp.int32),
            pltpu.VMEM((CH,), jnp.int32),
            pltpu.VMEM((CH, DS), jnp.float32),
            pltpu.VMEM((CH, DS), jnp.float32),
            pltpu.VMEM((CH, DS), jnp.float32),
            pltpu.VMEM((CH, DS), jnp.float32),
            pltpu.SemaphoreType.DMA,
            pltpu.SemaphoreType.DMA,
            pltpu.SemaphoreType.DMA,
            pltpu.SemaphoreType.DMA,
            pltpu.SemaphoreType.DMA,
            pltpu.SemaphoreType.DMA,
        ],
    )
    return kern(ta3, tb3, row, col, zh)


def _scc_body(cp128, rowh, colh, b1h, zc,
              cout,
              C, rblk, cblk, sidx0, sidx1, a0, b0, a1, b1, pb,
              sga0, sgb0, sga1, sgb1, ssc0, ssc1):
    c = lax.axis_index("c")
    t = lax.axis_index("s")
    rows0 = t * RPT

    pltpu.sync_copy(zc, C.at[pl.ds(rows0, RPT)])
    pltpu.sync_copy(b1h, pb)
    plsc.subcore_barrier()

    lane = jax.lax.iota(jnp.int32, L)
    m_cv = lane < 3
    m_deg = lane == 3
    bias1 = pb[...]
    cpc = cp128.at[c]

    def compute(aS, bS):
        def cj(j2, carry):
            for r in range(4):
                j = j2 * 4 + r
                sl = pl.ds(0, L)
                d = aS[j, sl] - bS[j, sl]
                sv = _silu(d + bias1)
                aS[j, sl] = jnp.where(m_cv, sv, jnp.where(m_deg, 1.0, 0.0))
            return carry
        lax.fori_loop(0, CCH // 4, cj, 0)

    w = t * NC + c
    _edge_pipeline(CCH, EPW // (BK * CCH), w * EPW, rowh, colh,
                   lambda idx: cpc.at[idx], lambda idx: cpc.at[idx], C,
                   rblk, cblk, sidx0, sidx1, a0, b0, a1, b1,
                   sga0, sgb0, sga1, sgb1, ssc0, ssc1, compute)

    plsc.subcore_barrier()
    pltpu.sync_copy(C.at[pl.ds(rows0, RPT)], cout.at[c, pl.ds(rows0, RPT)])


def _scc_call(cp128, row, col, b1v, zc):
    mesh = plsc.VectorSubcoreMesh(core_axis_name="c", subcore_axis_name="s")
    kern = pl.kernel(
        _scc_body,
        mesh=mesh,
        compiler_params=pltpu.CompilerParams(needs_layout_passes=False),
        out_type=[jax.ShapeDtypeStruct((NC, NP, DS), jnp.float32)],
        scratch_types=[
            pltpu.VMEM_SHARED((NP, DS), jnp.float32),
            pltpu.VMEM((BK * CCH,), jnp.int32),
            pltpu.VMEM((BK * CCH,), jnp.int32),
            pltpu.VMEM((CCH,), jnp.int32),
            pltpu.VMEM((CCH,), jnp.int32),
            pltpu.VMEM((CCH, DS), jnp.float32),
            pltpu.VMEM((CCH, DS), jnp.float32),
            pltpu.VMEM((CCH, DS), jnp.float32),
            pltpu.VMEM((CCH, DS), jnp.float32),
            pltpu.VMEM((L,), jnp.float32),
            pltpu.SemaphoreType.DMA,
            pltpu.SemaphoreType.DMA,
            pltpu.SemaphoreType.DMA,
            pltpu.SemaphoreType.DMA,
            pltpu.SemaphoreType.DMA,
            pltpu.SemaphoreType.DMA,
        ],
    )
    return kern(cp128, row, col, b1v, zc)


def _post_body(h_ref, c_ref, w2_ref, b2_ref, uW1_ref, ub1_ref, uW2_ref,
               ub2_ref, s_ref, v_ref, scal_ref, so_ref, vo_ref):
    cvd = c_ref[0, :, :L] + c_ref[1, :, :L]
    deg = cvd[:, 3:4]
    agg_s = (jnp.dot(h_ref[0], w2_ref[:DS, :], preferred_element_type=jnp.float32)
             + jnp.dot(h_ref[1], w2_ref[DS:, :], preferred_element_type=jnp.float32)
             + deg * b2_ref[...])
    tt = _silu(jnp.dot(agg_s, uW1_ref[...], preferred_element_type=jnp.float32)
               + ub1_ref[...])
    so_ref[...] = (s_ref[...] + jnp.dot(tt, uW2_ref[...],
                                        preferred_element_type=jnp.float32)
                   + ub2_ref[...])

    evW2 = scal_ref[0, 0]
    evb2 = scal_ref[0, 1]
    uvW1 = scal_ref[0, 2]
    uvb1 = scal_ref[0, 3]
    uvW2 = scal_ref[0, 4]
    uvb2 = scal_ref[0, 5]
    cv = cvd[:, :3]
    agg_v = cv * evW2 + deg * evb2
    vo_ref[...] = v_ref[...] + _silu(agg_v * uvW1 + uvb1) * uvW2 + uvb2


def _post(hout, cout, eW2, eb2_2d, uW1, ub1_2d, uW2, ub2_2d, s, v2, scal):
    return pl.pallas_call(
        _post_body,
        out_shape=[
            jax.ShapeDtypeStruct((N, DS), jnp.float32),
            jax.ShapeDtypeStruct((N, 3), jnp.float32),
        ],
    )(hout, cout, eW2, eb2_2d, uW1, ub1_2d, uW2, ub2_2d, s, v2, scal)


def kernel(s, v, coord, edge_index, eW1, eb1, eW2, eb2, evW1, evb1, evW2,
           evb2, uW1, ub1, uW2, ub2, uvW1, uvb1, uvW2, uvb2):
    row = edge_index[0]
    col = edge_index[1]

    ta3, tb3, cps = _prep(s, eW1, eb1[None, :], coord, evW1)

    cp128 = jnp.broadcast_to(jnp.pad(cps, ((0, 0), (0, DS - 3)))[None],
                             (NC, N, DS))
    b1v = jnp.full((L,), evb1[0], jnp.float32)
    zh = jnp.zeros((RPT, DS), jnp.float32)

    hout = _sch_call(ta3, tb3, row, col, zh)[0]
    cout = _scc_call(cp128, row, col, b1v, zh)[0]

    scal = jnp.stack([evW2[0, 0], evb2[0], uvW1[0, 0], uvb1[0], uvW2[0, 0],
                      uvb2[0], jnp.float32(0), jnp.float32(0)])[None, :]
    s_out, v_out = _post(hout[:, :N], cout[:, :N], eW2, eb2[None, :], uW1,
                         ub1[None, :], uW2, ub2[None, :], s, v.reshape(N, 3),
                         scal)
    return (s_out, v_out.reshape(N, 3, 1))


# speedup vs baseline: 4.9207x; 1.0289x over previous
import jax
import jax.numpy as jnp
from jax import lax
from jax.experimental import pallas as pl
from jax.experimental.pallas import tpu as pltpu
from jax.experimental.pallas import tpu_sc as plsc

N, E, DS = 10000, 320000, 128
NC, NS, L = 2, 16, 16
EPT = E // NS
EPW = E // (NC * NS)
CH = 80
CCH = 40
BK = 10
NP = 10240
RPT = NP // NS


def _silu(x):
    return x / (1.0 + jnp.exp(-x))


def _prep_body(s_ref, w1_ref, b1_ref, coord_ref, evw1_ref, ta_ref, tb_ref,
               cps_ref):
    s = s_ref[...]
    a = jnp.dot(s, w1_ref[:DS, :], preferred_element_type=jnp.float32) + b1_ref[...]
    b = jnp.dot(s, w1_ref[DS:, :], preferred_element_type=jnp.float32)
    ta_ref[0] = a[:, :DS]
    ta_ref[1] = a[:, DS:]
    tb_ref[0] = b[:, :DS]
    tb_ref[1] = b[:, DS:]
    cps_ref[...] = coord_ref[...] * evw1_ref[0, 0]


def _prep(s, eW1, eb1_2d, coord, evW1):
    out = jax.ShapeDtypeStruct((2, N, DS), jnp.float32)
    outc = jax.ShapeDtypeStruct((N, 3), jnp.float32)
    return pl.pallas_call(_prep_body, out_shape=[out, out, outc])(
        s, eW1, eb1_2d, coord, evW1)


def _edge_pipeline(ch, nblk, ebase, rowh, colh, src_a, src_b, acc,
                   rblk, cblk, sidx0, sidx1, a0, b0, a1, b1,
                   sga0, sgb0, sga1, sgb1, ssc0, ssc1, compute):
    def sidx_fill(k, sidx):
        def cp(j, carry):
            sidx[pl.ds(j * L, L)] = rblk[pl.ds(k * ch + j * L, L)]
            return carry
        lax.fori_loop(0, ch // L, cp, 0)

    def g_issue(k, aS, bS, sa, sb):
        ga = pltpu.async_copy(src_a(rblk.at[pl.ds(k * ch, ch)]), aS, sa)
        gb = pltpu.async_copy(src_b(cblk.at[pl.ds(k * ch, ch)]), bS, sb)
        return ga, gb

    def scat(sidx, aS, sem):
        return pltpu.async_copy(aS, acc.at[sidx], sem, add=True)

    def block(bi, carry):
        base = ebase + bi * (BK * ch)
        pltpu.sync_copy(rowh.at[pl.ds(base, BK * ch)], rblk)
        pltpu.sync_copy(colh.at[pl.ds(base, BK * ch)], cblk)
        g0 = g_issue(0, a0, b0, sga0, sgb0)
        sc0 = sc1 = None
        for p in range(BK // 2):
            k0 = 2 * p
            k1 = 2 * p + 1
            if sc1 is not None:
                sc1.wait()
            g1 = g_issue(k1, a1, b1, sga1, sgb1)
            g0[0].wait()
            g0[1].wait()
            compute(a0, b0)
            sidx_fill(k0, sidx0)
            sc0 = scat(sidx0, a0, ssc0)
            g1[0].wait()
            g1[1].wait()
            compute(a1, b1)
            sidx_fill(k1, sidx1)
            sc1 = scat(sidx1, a1, ssc1)
            if p < BK // 2 - 1:
                sc0.wait()
                g0 = g_issue(k0 + 2, a0, b0, sga0, sgb0)
        sc0.wait()
        sc1.wait()
        return carry
    lax.fori_loop(0, nblk, block, 0)


def _sch_body(ta, tb, rowh, colh, zh,
              hout,
              H, rblk, cblk, sidx0, sidx1, a0, b0, a1, b1,
              sga0, sgb0, sga1, sgb1, ssc0, ssc1):
    c = lax.axis_index("c")
    t = lax.axis_index("s")
    rows0 = t * RPT

    pltpu.sync_copy(zh, H.at[pl.ds(rows0, RPT)])
    plsc.subcore_barrier()

    tac = ta.at[c]
    tbc = tb.at[c]

    def compute(aS, bS):
        def cj(j2, carry):
            for r in range(4):
                j = j2 * 4 + r
                for k8 in range(DS // L):
                    sl = pl.ds(k8 * L, L)
                    x = aS[j, sl] + bS[j, sl]
                    aS[j, sl] = x / (1.0 + jnp.exp(-x))
            return carry
        lax.fori_loop(0, CH // 4, cj, 0)

    _edge_pipeline(CH, EPT // (BK * CH), t * EPT, rowh, colh,
                   lambda idx: tac.at[idx], lambda idx: tbc.at[idx], H,
                   rblk, cblk, sidx0, sidx1, a0, b0, a1, b1,
                   sga0, sgb0, sga1, sgb1, ssc0, ssc1, compute)

    plsc.subcore_barrier()
    pltpu.sync_copy(H.at[pl.ds(rows0, RPT)], hout.at[c, pl.ds(rows0, RPT)])


def _sch_call(ta3, tb3, row, col, zh):
    mesh = plsc.VectorSubcoreMesh(core_axis_name="c", subcore_axis_name="s")
    kern = pl.kernel(
        _sch_body,
        mesh=mesh,
        compiler_params=pltpu.CompilerParams(needs_layout_passes=False),
        out_type=[jax.ShapeDtypeStruct((NC, NP, DS), jnp.float32)],
        scratch_types=[
            pltpu.VMEM_SHARED((NP, DS), jnp.float32),
            pltpu.VMEM((BK * CH,), jnp.int32),
            pltpu.VMEM((BK * CH,), jnp.int32),
            pltpu.VMEM((CH,), jnp.int32),
            pltpu.VMEM((CH,), jnp.int32),
            pltpu.VMEM((CH, DS), jnp.float32),
            pltpu.VMEM((CH, DS), jnp.float32),
            pltpu.VMEM((CH, DS), jnp.float32),
            pltpu.VMEM((CH, DS), jnp.float32),
            pltpu.SemaphoreType.DMA,
            pltpu.SemaphoreType.DMA,
            pltpu.SemaphoreType.DMA,
            pltpu.SemaphoreType.DMA,
            pltpu.SemaphoreType.DMA,
            pltpu.SemaphoreType.DMA,
        ],
    )
    return kern(ta3, tb3, row, col, zh)


def _scc_body(cp128, rowh, colh, b1h, zc,
              cout,
              C, rblk, cblk, sidx0, sidx1, a0, b0, a1, b1, pb,
              sga0, sgb0, sga1, sgb1, ssc0, ssc1):
    c = lax.axis_index("c")
    t = lax.axis_index("s")
    rows0 = t * RPT

    pltpu.sync_copy(zc, C.at[pl.ds(rows0, RPT)])
    pltpu.sync_copy(b1h, pb)
    plsc.subcore_barrier()

    lane = jax.lax.iota(jnp.int32, L)
    m_cv = lane < 3
    m_deg = lane == 3
    bias1 = pb[...]
    cpc = cp128.at[c]

    def compute(aS, bS):
        def cj(j2, carry):
            for r in range(4):
                j = j2 * 4 + r
                sl = pl.ds(0, L)
                d = aS[j, sl] - bS[j, sl]
                sv = _silu(d + bias1)
                aS[j, sl] = jnp.where(m_cv, sv, jnp.where(m_deg, 1.0, 0.0))
            return carry
        lax.fori_loop(0, CCH // 4, cj, 0)

    w = t * NC + c
    _edge_pipeline(CCH, EPW // (BK * CCH), w * EPW, rowh, colh,
                   lambda idx: cpc.at[idx], lambda idx: cpc.at[idx], C,
                   rblk, cblk, sidx0, sidx1, a0, b0, a1, b1,
                   sga0, sgb0, sga1, sgb1, ssc0, ssc1, compute)

    plsc.subcore_barrier()
    pltpu.sync_copy(C.at[pl.ds(rows0, RPT)], cout.at[c, pl.ds(rows0, RPT)])


def _scc_call(cp128, row, col, b1v, zc):
    mesh = plsc.VectorSubcoreMesh(core_axis_name="c", subcore_axis_name="s")
    kern = pl.kernel(
        _scc_body,
        mesh=mesh,
        compiler_params=pltpu.CompilerParams(needs_layout_passes=False),
        out_type=[jax.ShapeDtypeStruct((NC, NP, DS), jnp.float32)],
        scratch_types=[
            pltpu.VMEM_SHARED((NP, DS), jnp.float32),
            pltpu.VMEM((BK * CCH,), jnp.int32),
            pltpu.VMEM((BK * CCH,), jnp.int32),
            pltpu.VMEM((CCH,), jnp.int32),
            pltpu.VMEM((CCH,), jnp.int32),
            pltpu.VMEM((CCH, DS), jnp.float32),
            pltpu.VMEM((CCH, DS), jnp.float32),
            pltpu.VMEM((CCH, DS), jnp.float32),
            pltpu.VMEM((CCH, DS), jnp.float32),
            pltpu.VMEM((L,), jnp.float32),
            pltpu.SemaphoreType.DMA,
            pltpu.SemaphoreType.DMA,
            pltpu.SemaphoreType.DMA,
            pltpu.SemaphoreType.DMA,
            pltpu.SemaphoreType.DMA,
            pltpu.SemaphoreType.DMA,
        ],
    )
    return kern(cp128, row, col, b1v, zc)


def _post_body(h_ref, c_ref, w2_ref, b2_ref, uW1_ref, ub1_ref, uW2_ref,
               ub2_ref, s_ref, v_ref, scal_ref, so_ref, vo_ref):
    cvd = c_ref[0, :, :L] + c_ref[1, :, :L]
    deg = cvd[:, 3:4]
    agg_s = (jnp.dot(h_ref[0], w2_ref[:DS, :], preferred_element_type=jnp.float32)
             + jnp.dot(h_ref[1], w2_ref[DS:, :], preferred_element_type=jnp.float32)
             + deg * b2_ref[...])
    tt = _silu(jnp.dot(agg_s, uW1_ref[...], preferred_element_type=jnp.float32)
               + ub1_ref[...])
    so_ref[...] = (s_ref[...] + jnp.dot(tt, uW2_ref[...],
                                        preferred_element_type=jnp.float32)
                   + ub2_ref[...])

    evW2 = scal_ref[0, 0]
    evb2 = scal_ref[0, 1]
    uvW1 = scal_ref[0, 2]
    uvb1 = scal_ref[0, 3]
    uvW2 = scal_ref[0, 4]
    uvb2 = scal_ref[0, 5]
    cv = cvd[:, :3]
    agg_v = cv * evW2 + deg * evb2
    vo_ref[...] = v_ref[...] + _silu(agg_v * uvW1 + uvb1) * uvW2 + uvb2


def _post(hout, cout, eW2, eb2_2d, uW1, ub1_2d, uW2, ub2_2d, s, v2, scal):
    return pl.pallas_call(
        _post_body,
        out_shape=[
            jax.ShapeDtypeStruct((N, DS), jnp.float32),
            jax.ShapeDtypeStruct((N, 3), jnp.float32),
        ],
    )(hout, cout, eW2, eb2_2d, uW1, ub1_2d, uW2, ub2_2d, s, v2, scal)


def kernel(s, v, coord, edge_index, eW1, eb1, eW2, eb2, evW1, evb1, evW2,
           evb2, uW1, ub1, uW2, ub2, uvW1, uvb1, uvW2, uvb2):
    row = edge_index[0]
    col = edge_index[1]

    ta3, tb3, cps = _prep(s, eW1, eb1[None, :], coord, evW1)

    cp128 = jnp.broadcast_to(jnp.pad(cps, ((0, 0), (0, DS - 3)))[None],
                             (NC, N, DS))
    b1v = jnp.full((L,), evb1[0], jnp.float32)
    zh = jnp.zeros((RPT, DS), jnp.float32)

    hout = _sch_call(ta3, tb3, row, col, zh)[0]
    cout = _scc_call(cp128, row, col, b1v, zh)[0]

    scal = jnp.stack([evW2[0, 0], evb2[0], uvW1[0, 0], uvb1[0], uvW2[0, 0],
                      uvb2[0], jnp.float32(0), jnp.float32(0)])[None, :]
    s_out, v_out = _post(hout[:, :N], cout[:, :N], eW2, eb2[None, :], uW1,
                         ub1[None, :], uW2, ub2[None, :], s, v.reshape(N, 3),
                         scal)
    return (s_out, v_out.reshape(N, 3, 1))
